# baseline (speedup 1.0000x reference)
import functools

import jax
import jax.numpy as jnp
from jax import lax
from jax.experimental import pallas as pl
from jax.experimental.pallas import tpu as pltpu
from jax.experimental.pallas import tpu_sc as plsc

N = 10000
E = 320000
F = 128
K1 = 5000
K2 = 2500

NP = 10240
NPR = NP // 128
NC = 2
NS = 16
NW = NC * NS
CH = 128
NCHUNK = 80
EPT = CH * NCHUNK
EP = EPT * NW
EPC = EP // CH
TPR = NP // NS

_SC_MESH = plsc.VectorSubcoreMesh(core_axis_name="c", subcore_axis_name="s")


@functools.partial(
    pl.kernel,
    out_type=(jax.ShapeDtypeStruct((NW, NP), jnp.float32),
              jax.ShapeDtypeStruct((NW, NP), jnp.float32)),
    mesh=_SC_MESH,
    compiler_params=pltpu.CompilerParams(needs_layout_passes=False),
    scratch_types=[
        pltpu.VMEM((NP,), jnp.float32),
        pltpu.VMEM((EPT,), jnp.int32),
        pltpu.VMEM((EPT,), jnp.int32),
        pltpu.VMEM((NP,), jnp.float32),
        pltpu.VMEM((NP,), jnp.float32),
    ],
)
def _deg_kernel(m_hbm, src_hbm, dst_hbm, dout_hbm, din_hbm,
                m_v, src_v, dst_v, ho_v, hi_v):
    c = lax.axis_index("c")
    s = lax.axis_index("s")
    wid = s * NC + c
    pltpu.sync_copy(m_hbm, m_v)
    base = wid * EPT
    pltpu.sync_copy(src_hbm.at[pl.ds(base, EPT)], src_v)
    pltpu.sync_copy(dst_hbm.at[pl.ds(base, EPT)], dst_v)

    zero16 = jnp.zeros((16,), jnp.float32)

    def zbody(j, carry):
        ho_v[pl.ds(j * 16, 16)] = zero16
        hi_v[pl.ds(j * 16, 16)] = zero16
        return carry

    lax.fori_loop(0, NP // 16, zbody, 0)

    def ebody(j, carry):
        isrc = src_v[pl.ds(j * 16, 16)]
        idst = dst_v[pl.ds(j * 16, 16)]
        ms = plsc.load_gather(m_v, [isrc])
        md = plsc.load_gather(m_v, [idst])
        v = ms * md
        plsc.addupdate_scatter(ho_v, [isrc], v)
        plsc.addupdate_scatter(hi_v, [idst], v)
        return carry

    lax.fori_loop(0, EPT // 16, ebody, 0)
    pltpu.sync_copy(ho_v, dout_hbm.at[wid])
    pltpu.sync_copy(hi_v, din_hbm.at[wid])


@functools.partial(
    pl.kernel,
    out_type=jax.ShapeDtypeStruct((NC, NP, F), jnp.float32),
    mesh=_SC_MESH,
    compiler_params=pltpu.CompilerParams(needs_layout_passes=False),
    scratch_types=[
        pltpu.VMEM((CH,), jnp.int32),
        pltpu.VMEM((CH,), jnp.int32),
        pltpu.VMEM((CH, F), jnp.float32),
        pltpu.VMEM_SHARED((NP, F), jnp.float32),
        pltpu.SemaphoreType.DMA,
    ],
)
def _edge_kernel(s_hbm, src_hbm, dst_hbm, z_hbm, out_hbm,
                 srcc, dstc, rows, acc, gsem):
    c = lax.axis_index("c")
    s = lax.axis_index("s")
    wid = s * NC + c
    pltpu.sync_copy(z_hbm, acc.at[pl.ds(s * TPR, TPR)])
    plsc.subcore_barrier()

    ebase = wid * EPT

    def body(g, carry):
        base = ebase + g * CH
        pltpu.sync_copy(src_hbm.at[pl.ds(base, CH)], srcc)
        pltpu.async_copy(s_hbm.at[srcc], rows, gsem).wait()
        pltpu.sync_copy(dst_hbm.at[pl.ds(base, CH)], dstc)
        pltpu.sync_copy(rows, acc.at[dstc], add=True)
        return carry

    lax.fori_loop(0, NCHUNK, body, 0)
    plsc.subcore_barrier()
    pltpu.sync_copy(acc.at[pl.ds(s * TPR, TPR)],
                    out_hbm.at[c, pl.ds(s * TPR, TPR)])


def _mm_body(x_ref, w_ref, dout_ref, o_ref):
    deg = jnp.sum(dout_ref[...], axis=0)
    scale = lax.rsqrt(jnp.maximum(deg, 1.0))
    o_ref[...] = jnp.dot(x_ref[...], w_ref[...],
                         preferred_element_type=jnp.float32) * scale[:, None]


def _mm(x, w, dout):
    return pl.pallas_call(
        _mm_body,
        out_shape=jax.ShapeDtypeStruct((NP, F), jnp.float32),
    )(x, w, dout)


def _act_body(a0_ref, a1_ref, din_ref, b_ref, p_ref, h_ref, s_ref):
    deg = jnp.sum(din_ref[...], axis=0)
    scale = lax.rsqrt(jnp.maximum(deg, 1.0))
    h = jnp.maximum((a0_ref[...] + a1_ref[...]) * scale[:, None] + b_ref[...],
                    0.0)
    h_ref[...] = h
    pn = lax.rsqrt(jnp.sum(p_ref[...] * p_ref[...]))
    s_ref[...] = jnp.tanh(
        jnp.dot(h, p_ref[...], preferred_element_type=jnp.float32) * pn)


def _act(a0, a1, din, b, p):
    return pl.pallas_call(
        _act_body,
        out_shape=(jax.ShapeDtypeStruct((NP, F), jnp.float32),
                   jax.ShapeDtypeStruct((NP, 1), jnp.float32)),
    )(a0, a1, din, b, p)


def _mask_body(s_ref, mprev_ref, m_ref, *, k):
    sb = s_ref[...] + 0.0
    bi = lax.bitcast_convert_type(sb, jnp.int32)
    mag = bi ^ jnp.int32(-2147483648)
    key = jnp.where(bi >= 0, bi, -mag)
    sent = jnp.int32(-1065353217)
    key = jnp.where(mprev_ref[...] > 0, key, sent)
    kk = jnp.int32(k)

    def vbody(_, lh):
        lo, hi = lh
        mid = (lo + hi + jnp.int32(1)) >> 1
        cnt = jnp.sum((key >= mid).astype(jnp.int32))
        pred = cnt >= kk
        return jnp.where(pred, mid, lo), jnp.where(pred, hi, mid - 1)

    v, _ = lax.fori_loop(0, 32, vbody, (sent, jnp.int32(1065353216)))

    cgt = jnp.sum((key > v).astype(jnp.int32))
    r = kk - cgt
    tie = key == v
    idx = (lax.broadcasted_iota(jnp.int32, (NPR, 128), 0) * 128
           + lax.broadcasted_iota(jnp.int32, (NPR, 128), 1))

    def ibody(_, lh):
        lo, hi = lh
        mid = (lo + hi) >> 1
        cnt = jnp.sum((tie & (idx < mid)).astype(jnp.int32))
        pred = cnt >= r
        return jnp.where(pred, lo, mid + 1), jnp.where(pred, mid, hi)

    i_thr, _ = lax.fori_loop(0, 14, ibody, (jnp.int32(0), jnp.int32(NP)))
    m_ref[...] = ((key > v) | (tie & (idx < i_thr))).astype(jnp.float32)


def _mask(score2d, mprev2d, k):
    return pl.pallas_call(
        functools.partial(_mask_body, k=k),
        out_shape=jax.ShapeDtypeStruct((NPR, 128), jnp.float32),
    )(score2d, mprev2d)


def _pool_body(h_ref, s_ref, m_ref, xn_ref, gmp_ref, gap_ref, *, k):
    xn = h_ref[...] * s_ref[...] * m_ref[...]
    xn_ref[...] = xn
    neg = jnp.float32(-3.4e38)
    gmp_ref[...] = jnp.max(jnp.where(m_ref[...] > 0, xn, neg),
                           axis=0, keepdims=True)
    gap_ref[...] = jnp.sum(xn, axis=0, keepdims=True) * jnp.float32(1.0 / k)


def _pool(h, s, m, k):
    return pl.pallas_call(
        functools.partial(_pool_body, k=k),
        out_shape=(jax.ShapeDtypeStruct((NP, F), jnp.float32),
                   jax.ShapeDtypeStruct((1, F), jnp.float32),
                   jax.ShapeDtypeStruct((1, F), jnp.float32)),
    )(h, s, m)


def _final_body(h_ref, m_ref, g1_ref, a1_ref, g2_ref, a2_ref, o_ref):
    m = m_ref[...]
    h = h_ref[...]
    neg = jnp.float32(-3.4e38)
    gmp3 = jnp.max(jnp.where(m > 0, h, neg), axis=0, keepdims=True)
    gap3 = jnp.sum(h * m, axis=0, keepdims=True) * jnp.float32(1.0 / K2)
    o_ref[...] = jnp.concatenate(
        [g1_ref[...] + g2_ref[...] + gmp3,
         a1_ref[...] + a2_ref[...] + gap3], axis=1)


def _final(h3, m2, gmp1, gap1, gmp2, gap2):
    return pl.pallas_call(
        _final_body,
        out_shape=jax.ShapeDtypeStruct((1, 2 * F), jnp.float32),
    )(h3, m2, gmp1, gap1, gmp2, gap2)


def kernel(x, edge_index, batch, W1, b1, W2, b2, W3, b3, p1, p2):
    del batch
    f32 = jnp.float32
    xp = jnp.pad(x.astype(f32), ((0, NP - N), (0, 0)))
    src = jnp.pad(edge_index[0], (0, EP - E), constant_values=NP - 1)
    dst = jnp.pad(edge_index[1], (0, EP - E), constant_values=NP - 1)
    m0 = (jnp.arange(NP, dtype=jnp.int32) < N).astype(f32)
    zb = jnp.zeros((TPR, F), f32)

    def layer(X, m, W, b, p):
        dout, din = _deg_kernel(m, src, dst)
        s_mat = _mm(X, W, dout)
        acc = _edge_kernel(s_mat, src, dst, zb)
        return _act(acc[0], acc[1], din, b.reshape(1, F), p.reshape(F, 1))

    h1, s1 = layer(xp, m0, W1, b1, p1)
    m1_2d = _mask(s1.reshape(NPR, 128), m0.reshape(NPR, 128), K1)
    m1 = m1_2d.reshape(NP, 1)
    X2, gmp1, gap1 = _pool(h1, s1, m1, K1)

    h2, s2 = layer(X2, m1.reshape(NP), W2, b2, p2)
    m2_2d = _mask(s2.reshape(NPR, 128), m1_2d, K2)
    m2 = m2_2d.reshape(NP, 1)
    X3, gmp2, gap2 = _pool(h2, s2, m2, K2)

    h3, _ = layer(X3, m2.reshape(NP), W3, b3, p2)
    return _final(h3, m2, gmp1, gap1, gmp2, gap2)


# speedup vs baseline: 1.4144x; 1.4144x over previous
import functools

import jax
import jax.numpy as jnp
from jax import lax
from jax.experimental import pallas as pl
from jax.experimental.pallas import tpu as pltpu
from jax.experimental.pallas import tpu_sc as plsc

N = 10000
E = 320000
F = 128
K1 = 5000
K2 = 2500

NP = 10240
NPR = NP // 128
NC = 2
NS = 16
NW = NC * NS
CH = 128
NCHUNK = 79
EPT = CH * NCHUNK
EP = EPT * NW
EPC = EP // CH
TPR = NP // NS

_SC_MESH = plsc.VectorSubcoreMesh(core_axis_name="c", subcore_axis_name="s")


@functools.partial(
    pl.kernel,
    out_type=(jax.ShapeDtypeStruct((NW, NP), jnp.float32),
              jax.ShapeDtypeStruct((NW, NP), jnp.float32)),
    mesh=_SC_MESH,
    compiler_params=pltpu.CompilerParams(needs_layout_passes=False),
    scratch_types=[
        pltpu.VMEM((NP,), jnp.float32),
        pltpu.VMEM((EPT,), jnp.int32),
        pltpu.VMEM((EPT,), jnp.int32),
        pltpu.VMEM((NP,), jnp.float32),
        pltpu.VMEM((NP,), jnp.float32),
    ],
)
def _deg_kernel(m_hbm, src_hbm, dst_hbm, dout_hbm, din_hbm,
                m_v, src_v, dst_v, ho_v, hi_v):
    c = lax.axis_index("c")
    s = lax.axis_index("s")
    wid = s * NC + c
    pltpu.sync_copy(m_hbm, m_v)
    base = wid * EPT
    pltpu.sync_copy(src_hbm.at[pl.ds(base, EPT)], src_v)
    pltpu.sync_copy(dst_hbm.at[pl.ds(base, EPT)], dst_v)

    zero16 = jnp.zeros((16,), jnp.float32)

    def zbody(j, carry):
        ho_v[pl.ds(j * 16, 16)] = zero16
        hi_v[pl.ds(j * 16, 16)] = zero16
        return carry

    lax.fori_loop(0, NP // 16, zbody, 0)

    def ebody(j, carry):
        isrc = src_v[pl.ds(j * 16, 16)]
        idst = dst_v[pl.ds(j * 16, 16)]
        ms = plsc.load_gather(m_v, [isrc])
        md = plsc.load_gather(m_v, [idst])
        v = ms * md
        plsc.addupdate_scatter(ho_v, [isrc], v)
        plsc.addupdate_scatter(hi_v, [idst], v)
        return carry

    lax.fori_loop(0, EPT // 16, ebody, 0)
    pltpu.sync_copy(ho_v, dout_hbm.at[wid])
    pltpu.sync_copy(hi_v, din_hbm.at[wid])


@functools.partial(
    pl.kernel,
    out_type=jax.ShapeDtypeStruct((NC, NP, F), jnp.float32),
    mesh=_SC_MESH,
    compiler_params=pltpu.CompilerParams(needs_layout_passes=False),
    scratch_types=[
        pltpu.VMEM((CH,), jnp.int32),
        pltpu.VMEM((CH,), jnp.int32),
        pltpu.VMEM((CH, F), jnp.float32),
        pltpu.VMEM_SHARED((NP, F), jnp.float32),
        pltpu.SemaphoreType.DMA,
    ],
)
def _edge_kernel(s_hbm, src_hbm, dst_hbm, z_hbm, out_hbm,
                 srcc, dstc, rows, acc, gsem):
    c = lax.axis_index("c")
    s = lax.axis_index("s")
    wid = s * NC + c
    pltpu.sync_copy(z_hbm, acc.at[pl.ds(s * TPR, TPR)])
    plsc.subcore_barrier()

    ebase = wid * EPT

    def body(g, carry):
        base = ebase + g * CH
        pltpu.sync_copy(src_hbm.at[pl.ds(base, CH)], srcc)
        pltpu.async_copy(s_hbm.at[srcc], rows, gsem).wait()
        pltpu.sync_copy(dst_hbm.at[pl.ds(base, CH)], dstc)
        pltpu.sync_copy(rows, acc.at[dstc], add=True)
        return carry

    lax.fori_loop(0, NCHUNK, body, 0)
    plsc.subcore_barrier()
    pltpu.sync_copy(acc.at[pl.ds(s * TPR, TPR)],
                    out_hbm.at[c, pl.ds(s * TPR, TPR)])


def _mm_body(x_ref, w_ref, dout_ref, o_ref):
    deg = jnp.sum(dout_ref[...], axis=0)
    scale = lax.rsqrt(jnp.maximum(deg, 1.0))
    o_ref[...] = jnp.dot(x_ref[...], w_ref[...],
                         preferred_element_type=jnp.float32) * scale[:, None]


def _mm(x, w, dout):
    return pl.pallas_call(
        _mm_body,
        out_shape=jax.ShapeDtypeStruct((NP, F), jnp.float32),
    )(x, w, dout)


def _act_body(a0_ref, a1_ref, din_ref, b_ref, p_ref, h_ref, s_ref):
    deg = jnp.sum(din_ref[...], axis=0)
    scale = lax.rsqrt(jnp.maximum(deg, 1.0))
    h = jnp.maximum((a0_ref[...] + a1_ref[...]) * scale[:, None] + b_ref[...],
                    0.0)
    h_ref[...] = h
    pn = lax.rsqrt(jnp.sum(p_ref[...] * p_ref[...]))
    s_ref[...] = jnp.tanh(
        jnp.dot(h, p_ref[...], preferred_element_type=jnp.float32) * pn)


def _act(a0, a1, din, b, p):
    return pl.pallas_call(
        _act_body,
        out_shape=(jax.ShapeDtypeStruct((NP, F), jnp.float32),
                   jax.ShapeDtypeStruct((NP, 1), jnp.float32)),
    )(a0, a1, din, b, p)


def _mask_body(s_ref, mprev_ref, m_ref, *, k):
    sb = s_ref[...] + 0.0
    bi = lax.bitcast_convert_type(sb, jnp.int32)
    mag = bi ^ jnp.int32(-2147483648)
    key = jnp.where(bi >= 0, bi, -mag)
    sent = jnp.int32(-1065353217)
    key = jnp.where(mprev_ref[...] > 0, key, sent)
    kk = jnp.int32(k)

    def vbody(_, lh):
        lo, hi = lh
        mid = (lo + hi + jnp.int32(1)) >> 1
        cnt = jnp.sum((key >= mid).astype(jnp.int32))
        pred = cnt >= kk
        return jnp.where(pred, mid, lo), jnp.where(pred, hi, mid - 1)

    v, _ = lax.fori_loop(0, 32, vbody, (sent, jnp.int32(1065353216)))

    cgt = jnp.sum((key > v).astype(jnp.int32))
    r = kk - cgt
    tie = key == v
    idx = (lax.broadcasted_iota(jnp.int32, (NPR, 128), 0) * 128
           + lax.broadcasted_iota(jnp.int32, (NPR, 128), 1))

    def ibody(_, lh):
        lo, hi = lh
        mid = (lo + hi) >> 1
        cnt = jnp.sum((tie & (idx < mid)).astype(jnp.int32))
        pred = cnt >= r
        return jnp.where(pred, lo, mid + 1), jnp.where(pred, mid, hi)

    i_thr, _ = lax.fori_loop(0, 14, ibody, (jnp.int32(0), jnp.int32(NP)))
    m_ref[...] = ((key > v) | (tie & (idx < i_thr))).astype(jnp.float32)


def _mask(score2d, mprev2d, k):
    return pl.pallas_call(
        functools.partial(_mask_body, k=k),
        out_shape=jax.ShapeDtypeStruct((NPR, 128), jnp.float32),
    )(score2d, mprev2d)


def _pool_body(h_ref, s_ref, m_ref, xn_ref, gmp_ref, gap_ref, *, k):
    xn = h_ref[...] * s_ref[...] * m_ref[...]
    xn_ref[...] = xn
    neg = jnp.float32(-3.4e38)
    gmp_ref[...] = jnp.max(jnp.where(m_ref[...] > 0, xn, neg),
                           axis=0, keepdims=True)
    gap_ref[...] = jnp.sum(xn, axis=0, keepdims=True) * jnp.float32(1.0 / k)


def _pool(h, s, m, k):
    return pl.pallas_call(
        functools.partial(_pool_body, k=k),
        out_shape=(jax.ShapeDtypeStruct((NP, F), jnp.float32),
                   jax.ShapeDtypeStruct((1, F), jnp.float32),
                   jax.ShapeDtypeStruct((1, F), jnp.float32)),
    )(h, s, m)


def _final_body(h_ref, m_ref, g1_ref, a1_ref, g2_ref, a2_ref, o_ref):
    m = m_ref[...]
    h = h_ref[...]
    neg = jnp.float32(-3.4e38)
    gmp3 = jnp.max(jnp.where(m > 0, h, neg), axis=0, keepdims=True)
    gap3 = jnp.sum(h * m, axis=0, keepdims=True) * jnp.float32(1.0 / K2)
    o_ref[...] = jnp.concatenate(
        [g1_ref[...] + g2_ref[...] + gmp3,
         a1_ref[...] + a2_ref[...] + gap3], axis=1)


def _final(h3, m2, gmp1, gap1, gmp2, gap2):
    return pl.pallas_call(
        _final_body,
        out_shape=jax.ShapeDtypeStruct((1, 2 * F), jnp.float32),
    )(h3, m2, gmp1, gap1, gmp2, gap2)


def kernel(x, edge_index, batch, W1, b1, W2, b2, W3, b3, p1, p2):
    del batch
    f32 = jnp.float32
    xp = jnp.pad(x.astype(f32), ((0, NP - N), (0, 0)))
    src = jnp.pad(edge_index[0], (0, EP - E), constant_values=NP - 1)
    dst = jnp.pad(edge_index[1], (0, EP - E), constant_values=NP - 1)
    m0 = (jnp.arange(NP, dtype=jnp.int32) < N).astype(f32)
    zb = jnp.zeros((TPR, F), f32)

    def layer(X, m, W, b, p):
        dout, din = _deg_kernel(m, src, dst)
        s_mat = _mm(X, W, dout)
        acc = _edge_kernel(s_mat, src, dst, zb)
        return _act(acc[0], acc[1], din, b.reshape(1, F), p.reshape(F, 1))

    h1, s1 = layer(xp, m0, W1, b1, p1)
    m1_2d = _mask(s1.reshape(NPR, 128), m0.reshape(NPR, 128), K1)
    m1 = m1_2d.reshape(NP, 1)
    X2, gmp1, gap1 = _pool(h1, s1, m1, K1)

    h2, s2 = layer(X2, m1.reshape(NP), W2, b2, p2)
    m2_2d = _mask(s2.reshape(NPR, 128), m1_2d, K2)
    m2 = m2_2d.reshape(NP, 1)
    X3, gmp2, gap2 = _pool(h2, s2, m2, K2)

    h3, _ = layer(X3, m2.reshape(NP), W3, b3, p2)
    return _final(h3, m2, gmp1, gap1, gmp2, gap2)


# speedup vs baseline: 2.0463x; 1.4467x over previous
import functools

import jax
import jax.numpy as jnp
from jax import lax
from jax.experimental import pallas as pl
from jax.experimental.pallas import tpu as pltpu
from jax.experimental.pallas import tpu_sc as plsc

N = 10000
E = 320000
F = 128
K1 = 5000
K2 = 2500

NP = 10240
NPR = NP // 128
NC = 2
NS = 16
NW = NC * NS
CH = 128
NCHUNK = 79
EPT = CH * NCHUNK
ERT = 10000
TSC = 88
TSTRIDE = CH * TSC
EP = TSTRIDE * NW
EPC = EP // CH
TPR = NP // NS

_SC_MESH = plsc.VectorSubcoreMesh(core_axis_name="c", subcore_axis_name="s")


@functools.partial(
    pl.kernel,
    out_type=(jax.ShapeDtypeStruct((NW, NP), jnp.float32),
              jax.ShapeDtypeStruct((NW, NP), jnp.float32)),
    mesh=_SC_MESH,
    compiler_params=pltpu.CompilerParams(needs_layout_passes=False),
    scratch_types=[
        pltpu.VMEM((NP,), jnp.float32),
        pltpu.VMEM((EPT,), jnp.int32),
        pltpu.VMEM((EPT,), jnp.int32),
        pltpu.VMEM((NP,), jnp.float32),
        pltpu.VMEM((NP,), jnp.float32),
    ],
)
def _deg_kernel(m_hbm, src_hbm, dst_hbm, dout_hbm, din_hbm,
                m_v, src_v, dst_v, ho_v, hi_v):
    c = lax.axis_index("c")
    s = lax.axis_index("s")
    wid = s * NC + c
    pltpu.sync_copy(m_hbm, m_v)
    base = wid * TSTRIDE
    pltpu.sync_copy(src_hbm.at[pl.ds(base, EPT)], src_v)
    pltpu.sync_copy(dst_hbm.at[pl.ds(base, EPT)], dst_v)

    zero16 = jnp.zeros((16,), jnp.float32)

    def zbody(j, carry):
        ho_v[pl.ds(j * 16, 16)] = zero16
        hi_v[pl.ds(j * 16, 16)] = zero16
        return carry

    lax.fori_loop(0, NP // 16, zbody, 0)

    def ebody(j, carry):
        isrc = src_v[pl.ds(j * 16, 16)]
        idst = dst_v[pl.ds(j * 16, 16)]
        ms = plsc.load_gather(m_v, [isrc])
        md = plsc.load_gather(m_v, [idst])
        v = ms * md
        plsc.addupdate_scatter(ho_v, [isrc], v)
        plsc.addupdate_scatter(hi_v, [idst], v)
        return carry

    lax.fori_loop(0, EPT // 16, ebody, 0)
    pltpu.sync_copy(ho_v, dout_hbm.at[wid])
    pltpu.sync_copy(hi_v, din_hbm.at[wid])


@functools.partial(
    pl.kernel,
    out_type=jax.ShapeDtypeStruct((NC, NP, F), jnp.float32),
    mesh=_SC_MESH,
    compiler_params=pltpu.CompilerParams(needs_layout_passes=False),
    scratch_types=[
        pltpu.VMEM((NCHUNK + 1, CH), jnp.int32),
        pltpu.VMEM((CH,), jnp.int32),
        pltpu.VMEM((CH,), jnp.int32),
        pltpu.VMEM((CH, F), jnp.float32),
        pltpu.VMEM((CH, F), jnp.float32),
        pltpu.VMEM_SHARED((NP, F), jnp.float32),
        pltpu.SemaphoreType.DMA,
        pltpu.SemaphoreType.DMA,
        pltpu.SemaphoreType.DMA,
    ],
)
def _edge_kernel(s_hbm, src_hbm, dst_hbm, z_hbm, out_hbm,
                 dstv, srcc0, srcc1, rows0, rows1, acc, gsem, ssem, isem):
    c = lax.axis_index("c")
    s = lax.axis_index("s")
    wid = s * NC + c
    pltpu.sync_copy(dst_hbm.at[pl.ds(wid * TSC, NCHUNK + 1)], dstv)
    pltpu.sync_copy(z_hbm, acc.at[pl.ds(s * TPR, TPR)])
    plsc.subcore_barrier()

    rows = (rows0, rows1)
    srcc = (srcc0, srcc1)
    ebase = wid * TSTRIDE
    pltpu.sync_copy(src_hbm.at[pl.ds(ebase, CH)], srcc0)
    pltpu.async_copy(s_hbm.at[srcc0], rows0, gsem).wait()
    pltpu.async_copy(src_hbm.at[pl.ds(ebase + CH, CH)], srcc1, isem)

    def outer(i, carry):
        for b in (0, 1):
            g = 2 * i + b
            gn2 = jnp.where(g + 2 < NCHUNK, g + 2, 0)
            pltpu.make_async_copy(
                src_hbm.at[pl.ds(ebase, CH)], srcc[1 - b], isem).wait()
            gd = pltpu.async_copy(s_hbm.at[srcc[1 - b]], rows[1 - b], gsem)
            pltpu.async_copy(src_hbm.at[pl.ds(ebase + gn2 * CH, CH)],
                             srcc[b], isem)
            sd = pltpu.async_copy(rows[b], acc.at[dstv.at[g]], ssem, add=True)
            sd.wait()
            gd.wait()
        return carry

    lax.fori_loop(0, NCHUNK // 2, outer, 0)
    pltpu.sync_copy(rows0, acc.at[dstv.at[NCHUNK - 1]], add=True)
    pltpu.make_async_copy(src_hbm.at[pl.ds(ebase, CH)], srcc1, isem).wait()
    plsc.subcore_barrier()
    pltpu.sync_copy(acc.at[pl.ds(s * TPR, TPR)],
                    out_hbm.at[c, pl.ds(s * TPR, TPR)])


def _mm_body(x_ref, w_ref, dout_ref, o_ref):
    deg = jnp.sum(dout_ref[...], axis=0)
    scale = lax.rsqrt(jnp.maximum(deg, 1.0))
    o_ref[...] = jnp.dot(x_ref[...], w_ref[...],
                         preferred_element_type=jnp.float32) * scale[:, None]


def _mm(x, w, dout):
    return pl.pallas_call(
        _mm_body,
        out_shape=jax.ShapeDtypeStruct((NP, F), jnp.float32),
    )(x, w, dout)


def _act_body(a0_ref, a1_ref, din_ref, b_ref, p_ref, h_ref, s_ref):
    deg = jnp.sum(din_ref[...], axis=0)
    scale = lax.rsqrt(jnp.maximum(deg, 1.0))
    h = jnp.maximum((a0_ref[...] + a1_ref[...]) * scale[:, None] + b_ref[...],
                    0.0)
    h_ref[...] = h
    pn = lax.rsqrt(jnp.sum(p_ref[...] * p_ref[...]))
    s_ref[...] = jnp.tanh(
        jnp.dot(h, p_ref[...], preferred_element_type=jnp.float32) * pn)


def _act(a0, a1, din, b, p):
    return pl.pallas_call(
        _act_body,
        out_shape=(jax.ShapeDtypeStruct((NP, F), jnp.float32),
                   jax.ShapeDtypeStruct((NP, 1), jnp.float32)),
    )(a0, a1, din, b, p)


def _mask_body(s_ref, mprev_ref, m_ref, *, k):
    sb = s_ref[...] + 0.0
    bi = lax.bitcast_convert_type(sb, jnp.int32)
    mag = bi ^ jnp.int32(-2147483648)
    key = jnp.where(bi >= 0, bi, -mag)
    sent = jnp.int32(-1065353217)
    key = jnp.where(mprev_ref[...] > 0, key, sent)
    kk = jnp.int32(k)

    def vbody(_, lh):
        lo, hi = lh
        mid = (lo + hi + jnp.int32(1)) >> 1
        cnt = jnp.sum((key >= mid).astype(jnp.int32))
        pred = cnt >= kk
        return jnp.where(pred, mid, lo), jnp.where(pred, hi, mid - 1)

    v, _ = lax.fori_loop(0, 32, vbody, (sent, jnp.int32(1065353216)))

    cgt = jnp.sum((key > v).astype(jnp.int32))
    r = kk - cgt
    tie = key == v
    idx = (lax.broadcasted_iota(jnp.int32, (NPR, 128), 0) * 128
           + lax.broadcasted_iota(jnp.int32, (NPR, 128), 1))

    def ibody(_, lh):
        lo, hi = lh
        mid = (lo + hi) >> 1
        cnt = jnp.sum((tie & (idx < mid)).astype(jnp.int32))
        pred = cnt >= r
        return jnp.where(pred, lo, mid + 1), jnp.where(pred, mid, hi)

    i_thr, _ = lax.fori_loop(0, 14, ibody, (jnp.int32(0), jnp.int32(NP)))
    m_ref[...] = ((key > v) | (tie & (idx < i_thr))).astype(jnp.float32)


def _mask(score2d, mprev2d, k):
    return pl.pallas_call(
        functools.partial(_mask_body, k=k),
        out_shape=jax.ShapeDtypeStruct((NPR, 128), jnp.float32),
    )(score2d, mprev2d)


def _pool_body(h_ref, s_ref, m_ref, xn_ref, gmp_ref, gap_ref, *, k):
    xn = h_ref[...] * s_ref[...] * m_ref[...]
    xn_ref[...] = xn
    neg = jnp.float32(-3.4e38)
    gmp_ref[...] = jnp.max(jnp.where(m_ref[...] > 0, xn, neg),
                           axis=0, keepdims=True)
    gap_ref[...] = jnp.sum(xn, axis=0, keepdims=True) * jnp.float32(1.0 / k)


def _pool(h, s, m, k):
    return pl.pallas_call(
        functools.partial(_pool_body, k=k),
        out_shape=(jax.ShapeDtypeStruct((NP, F), jnp.float32),
                   jax.ShapeDtypeStruct((1, F), jnp.float32),
                   jax.ShapeDtypeStruct((1, F), jnp.float32)),
    )(h, s, m)


def _final_body(h_ref, m_ref, g1_ref, a1_ref, g2_ref, a2_ref, o_ref):
    m = m_ref[...]
    h = h_ref[...]
    neg = jnp.float32(-3.4e38)
    gmp3 = jnp.max(jnp.where(m > 0, h, neg), axis=0, keepdims=True)
    gap3 = jnp.sum(h * m, axis=0, keepdims=True) * jnp.float32(1.0 / K2)
    o_ref[...] = jnp.concatenate(
        [g1_ref[...] + g2_ref[...] + gmp3,
         a1_ref[...] + a2_ref[...] + gap3], axis=1)


def _final(h3, m2, gmp1, gap1, gmp2, gap2):
    return pl.pallas_call(
        _final_body,
        out_shape=jax.ShapeDtypeStruct((1, 2 * F), jnp.float32),
    )(h3, m2, gmp1, gap1, gmp2, gap2)


def kernel(x, edge_index, batch, W1, b1, W2, b2, W3, b3, p1, p2):
    del batch
    f32 = jnp.float32
    xp = jnp.pad(x.astype(f32), ((0, NP - N), (0, 0)))
    src = jnp.pad(edge_index[0].reshape(NW, ERT),
                  ((0, 0), (0, TSTRIDE - ERT)),
                  constant_values=NP - 1).reshape(-1)
    dst = jnp.pad(edge_index[1].reshape(NW, ERT),
                  ((0, 0), (0, TSTRIDE - ERT)),
                  constant_values=NP - 1).reshape(-1)
    m0 = (jnp.arange(NP, dtype=jnp.int32) < N).astype(f32)
    zb = jnp.zeros((TPR, F), f32)

    dst2d = dst.reshape(EPC, CH)

    def layer(X, m, W, b, p):
        dout, din = _deg_kernel(m, src, dst)
        s_mat = _mm(X, W, dout)
        acc = _edge_kernel(s_mat, src, dst2d, zb)
        return _act(acc[0], acc[1], din, b.reshape(1, F), p.reshape(F, 1))

    h1, s1 = layer(xp, m0, W1, b1, p1)
    m1_2d = _mask(s1.reshape(NPR, 128), m0.reshape(NPR, 128), K1)
    m1 = m1_2d.reshape(NP, 1)
    X2, gmp1, gap1 = _pool(h1, s1, m1, K1)

    h2, s2 = layer(X2, m1.reshape(NP), W2, b2, p2)
    m2_2d = _mask(s2.reshape(NPR, 128), m1_2d, K2)
    m2 = m2_2d.reshape(NP, 1)
    X3, gmp2, gap2 = _pool(h2, s2, m2, K2)

    h3, _ = layer(X3, m2.reshape(NP), W3, b3, p2)
    return _final(h3, m2, gmp1, gap1, gmp2, gap2)
